# baseline (speedup 1.0000x reference)
import functools

import jax
import jax.numpy as jnp
from jax import lax
from jax.experimental import pallas as pl
from jax.experimental.pallas import tpu as pltpu
from jax.experimental.pallas import tpu_sc as plsc

N = 32768
NUM_MIX = 8
NC = 2
NS = 16
LANES = 16
NW = NC * NS
CHUNK = N // NW
NSLICE = CHUNK // LANES

_CT_F = NUM_MIX * NUM_MIX
_A_OFF = 0
_W2_OFF = 8
_B2_OFF = _W2_OFF + 64
_W3_OFF = _B2_OFF + 8
_B3_OFF = _W3_OFF + 8
_WP_ROWS = _B3_OFF + 1
_NET_F = _CT_F + _WP_ROWS * LANES
_WTOT = 2 * _NET_F


def _sc_body(obs_hbm, z_hbm, w_hbm, mean_hbm, std_hbm,
             obs_v, z_v, w_v, out_v, sem):
    wid = lax.axis_index("c") * NS + lax.axis_index("s")
    base = wid * CHUNK

    c1 = pltpu.async_copy(obs_hbm.at[pl.ds(base, CHUNK)], obs_v, sem)
    c2 = pltpu.async_copy(z_hbm.at[pl.ds(base, CHUNK)], z_v, sem)
    c3 = pltpu.async_copy(w_hbm, w_v, sem)
    c1.wait()
    c2.wait()
    c3.wait()

    def run_net(nb, carry):
        ctb = nb * _NET_F
        wpb = ctb + _CT_F
        ob = nb * CHUNK
        is_std = nb == 1

        def row(r):
            return w_v[pl.ds(wpb + r * LANES, LANES)]

        a1 = [row(_A_OFF + j) for j in range(NUM_MIX)]
        b2 = [row(_B2_OFF + i) for i in range(NUM_MIX)]
        w3 = [row(_W3_OFF + i) for i in range(NUM_MIX)]
        b3 = row(_B3_OFF)

        @plsc.parallel_loop(0, NSLICE, unroll=1)
        def slice_body(s):
            o = s * LANES
            obs16 = obs_v[pl.ds(o, LANES)]
            z8 = (z_v[pl.ds(o, LANES)] * NUM_MIX) + ctb

            s1 = []
            for j in range(NUM_MIX):
                cz = plsc.load_gather(w_v, [z8 + j])
                t = jnp.exp(obs16 * a1[j] + cz)
                s1.append(1.0 / (t + 1.0))

            out = b3
            for i in range(NUM_MIX):
                acc = b2[i]
                for j in range(NUM_MIX):
                    acc = acc + row(_W2_OFF + i * NUM_MIX + j) * s1[j]
                t = jnp.exp(acc)
                out = out + w3[i] * (1.0 / (t + 1.0))

            out_v[pl.ds(ob + o, LANES)] = jnp.where(is_std, jnp.exp(out), out)

        return carry

    lax.fori_loop(0, 2, run_net, 0)

    pltpu.sync_copy(out_v.at[pl.ds(0, CHUNK)], mean_hbm.at[pl.ds(base, CHUNK)])
    pltpu.sync_copy(out_v.at[pl.ds(CHUNK, CHUNK)],
                    std_hbm.at[pl.ds(base, CHUNK)])


def _scratch_types():
    return [
        pltpu.VMEM((CHUNK,), jnp.float32),
        pltpu.VMEM((CHUNK,), jnp.int32),
        pltpu.VMEM((_WTOT,), jnp.float32),
        pltpu.VMEM((2 * CHUNK,), jnp.float32),
        pltpu.SemaphoreType.DMA,
    ]


@functools.cache
def _sc_call():
    return functools.partial(
        pl.kernel,
        out_type=(
            jax.ShapeDtypeStruct((N,), jnp.float32),
            jax.ShapeDtypeStruct((N,), jnp.float32),
        ),
        mesh=plsc.VectorSubcoreMesh(
            core_axis_name="c", subcore_axis_name="s",
            num_cores=NC, num_subcores=NS,
        ),
        scratch_types=_scratch_types(),
        compiler_params=pltpu.CompilerParams(needs_layout_passes=False),
    )(_sc_body)


def _pack_net(W1, b1, W2, b2, W3, b3):
    a1 = -2.0 * W1[:, 0]
    ct = (-2.0 * (W1[:, 1:].T + b1[None, :])).reshape(-1)
    w2p = -4.0 * W2
    b2p = -2.0 * b2 + 2.0 * W2.sum(axis=1)
    w3p = 2.0 * W3[0]
    b3p = b3 - W3[0].sum()
    return jnp.concatenate([
        ct,
        jnp.repeat(a1, LANES),
        jnp.repeat(w2p.reshape(-1), LANES),
        jnp.repeat(b2p, LANES),
        jnp.repeat(w3p, LANES),
        jnp.repeat(b3p, LANES),
    ])


def kernel(obs, k, z, mW1, mb1, mW2, mb2, mW3, mb3,
           sW1, sb1, sW2, sb2, sW3, sb3):
    del k
    wbuf = jnp.concatenate([
        _pack_net(mW1, mb1, mW2, mb2, mW3, mb3),
        _pack_net(sW1, sb1, sW2, sb2, sW3, sb3),
    ])
    mean, std = _sc_call()(obs, z.astype(jnp.int32), wbuf)
    return mean, std


# speedup vs baseline: 1.0009x; 1.0009x over previous
import functools

import jax
import jax.numpy as jnp
from jax import lax
from jax.experimental import pallas as pl
from jax.experimental.pallas import tpu as pltpu
from jax.experimental.pallas import tpu_sc as plsc

N = 32768
NUM_MIX = 8
NC = 2
NS = 16
LANES = 16
NW = NC * NS
CHUNK = N // NW
NSLICE = CHUNK // LANES

_CT_F = NUM_MIX * NUM_MIX
_A_OFF = 0
_W2_OFF = 8
_B2_OFF = _W2_OFF + 64
_W3_OFF = _B2_OFF + 8
_B3_OFF = _W3_OFF + 8
_WP_ROWS = _B3_OFF + 1
_NET_F = _CT_F + _WP_ROWS * LANES
_WTOT = 2 * _NET_F


def _sc_body(obs_hbm, z_hbm, w_hbm, mean_hbm, std_hbm,
             obs_v, z_v, w_v, out_v, sem):
    wid = lax.axis_index("c") * NS + lax.axis_index("s")
    base = wid * CHUNK

    c1 = pltpu.async_copy(obs_hbm.at[pl.ds(base, CHUNK)], obs_v, sem)
    c2 = pltpu.async_copy(z_hbm.at[pl.ds(base, CHUNK)], z_v, sem)
    c3 = pltpu.async_copy(w_hbm, w_v, sem)
    c1.wait()
    c2.wait()
    c3.wait()

    def run_net(nb, carry):
        ctb = nb * _NET_F
        wpb = ctb + _CT_F
        ob = nb * CHUNK
        is_std = nb == 1

        def row(r):
            return w_v[pl.ds(wpb + r * LANES, LANES)]

        a1 = [row(_A_OFF + j) for j in range(NUM_MIX)]
        b2 = [row(_B2_OFF + i) for i in range(NUM_MIX)]
        w3 = [row(_W3_OFF + i) for i in range(NUM_MIX)]
        b3 = row(_B3_OFF)

        @plsc.parallel_loop(0, NSLICE, unroll=1)
        def slice_body(s):
            o = s * LANES
            obs16 = obs_v[pl.ds(o, LANES)]
            z8 = (z_v[pl.ds(o, LANES)] * NUM_MIX) + ctb

            s1 = []
            for j in range(NUM_MIX):
                cz = plsc.load_gather(w_v, [z8 + j])
                t = jnp.exp(obs16 * a1[j] + cz)
                s1.append(1.0 / (t + 1.0))

            out = b3
            for i in range(NUM_MIX):
                acc = b2[i]
                for j in range(NUM_MIX):
                    acc = acc + row(_W2_OFF + i * NUM_MIX + j) * s1[j]
                t = jnp.exp(acc)
                out = out + w3[i] * (1.0 / (t + 1.0))

            out_v[pl.ds(ob + o, LANES)] = jnp.where(is_std, jnp.exp(out), out)

        return carry

    lax.fori_loop(0, 2, run_net, 0)

    pltpu.sync_copy(out_v.at[pl.ds(0, CHUNK)], mean_hbm.at[pl.ds(base, CHUNK)])
    pltpu.sync_copy(out_v.at[pl.ds(CHUNK, CHUNK)],
                    std_hbm.at[pl.ds(base, CHUNK)])


def _scratch_types():
    return [
        pltpu.VMEM((CHUNK,), jnp.float32),
        pltpu.VMEM((CHUNK,), jnp.int32),
        pltpu.VMEM((_WTOT,), jnp.float32),
        pltpu.VMEM((2 * CHUNK,), jnp.float32),
        pltpu.SemaphoreType.DMA,
    ]


@functools.cache
def _sc_call():
    return functools.partial(
        pl.kernel,
        out_type=(
            jax.ShapeDtypeStruct((N,), jnp.float32),
            jax.ShapeDtypeStruct((N,), jnp.float32),
        ),
        mesh=plsc.VectorSubcoreMesh(
            core_axis_name="c", subcore_axis_name="s",
            num_cores=NC, num_subcores=NS,
        ),
        scratch_types=_scratch_types(),
        compiler_params=pltpu.CompilerParams(
            needs_layout_passes=False,
            disable_bounds_checks=True,
            disable_semaphore_checks=True,
        ),
    )(_sc_body)


def _pack_net(W1, b1, W2, b2, W3, b3):
    a1 = -2.0 * W1[:, 0]
    ct = (-2.0 * (W1[:, 1:].T + b1[None, :])).reshape(-1)
    w2p = -4.0 * W2
    b2p = -2.0 * b2 + 2.0 * W2.sum(axis=1)
    w3p = 2.0 * W3[0]
    b3p = b3 - W3[0].sum()
    return jnp.concatenate([
        ct,
        jnp.repeat(a1, LANES),
        jnp.repeat(w2p.reshape(-1), LANES),
        jnp.repeat(b2p, LANES),
        jnp.repeat(w3p, LANES),
        jnp.repeat(b3p, LANES),
    ])


def kernel(obs, k, z, mW1, mb1, mW2, mb2, mW3, mb3,
           sW1, sb1, sW2, sb2, sW3, sb3):
    del k
    wbuf = jnp.concatenate([
        _pack_net(mW1, mb1, mW2, mb2, mW3, mb3),
        _pack_net(sW1, sb1, sW2, sb2, sW3, sb3),
    ])
    mean, std = _sc_call()(obs, z.astype(jnp.int32), wbuf)
    return mean, std


# speedup vs baseline: 1.0293x; 1.0284x over previous
import functools

import jax
import jax.numpy as jnp
from jax import lax
from jax.experimental import pallas as pl
from jax.experimental.pallas import tpu as pltpu
from jax.experimental.pallas import tpu_sc as plsc

N = 32768
NUM_MIX = 8
NC = 2
NS = 16
LANES = 16
NW = NC * NS
CHUNK = N // NW
NSLICE = CHUNK // LANES

_CT_F = NUM_MIX * NUM_MIX
_A_OFF = 0
_W2_OFF = 8
_B2_OFF = _W2_OFF + 64
_W3_OFF = _B2_OFF + 8
_B3_OFF = _W3_OFF + 8
_WP_ROWS = _B3_OFF + 1
_NET_F = _CT_F + _WP_ROWS
_ROWS_F = _WP_ROWS * LANES


def _sc_body(obs_hbm, z_hbm, cw_hbm, mean_hbm, std_hbm,
             obs_v, z_v, cw_v, w_v, out_v, sem):
    wid = lax.axis_index("c") * NS + lax.axis_index("s")
    base = wid * CHUNK

    c1 = pltpu.async_copy(obs_hbm.at[pl.ds(base, CHUNK)], obs_v, sem)
    c2 = pltpu.async_copy(z_hbm.at[pl.ds(base, CHUNK)], z_v, sem)
    c3 = pltpu.async_copy(cw_hbm, cw_v, sem)
    c1.wait()
    c2.wait()
    c3.wait()

    @plsc.parallel_loop(0, _WP_ROWS, unroll=1)
    def expand(r):
        for n in range(2):
            idx = jnp.full((LANES,), n * _NET_F + _CT_F + r, jnp.int32)
            w_v[pl.ds((n * _WP_ROWS + r) * LANES, LANES)] = (
                plsc.load_gather(cw_v, [idx]))

    def run_net(nb, carry):
        ctb = nb * _NET_F
        wrb = nb * _ROWS_F
        ob = nb * CHUNK
        is_std = nb == 1

        def row(r):
            return w_v[pl.ds(wrb + r * LANES, LANES)]

        a1 = [row(_A_OFF + j) for j in range(NUM_MIX)]
        b2 = [row(_B2_OFF + i) for i in range(NUM_MIX)]
        w3 = [row(_W3_OFF + i) for i in range(NUM_MIX)]
        b3 = row(_B3_OFF)

        @plsc.parallel_loop(0, NSLICE, unroll=1)
        def slice_body(s):
            o = s * LANES
            obs16 = obs_v[pl.ds(o, LANES)]
            z8 = (z_v[pl.ds(o, LANES)] * NUM_MIX) + ctb

            s1 = []
            for j in range(NUM_MIX):
                cz = plsc.load_gather(cw_v, [z8 + j])
                t = jnp.exp(obs16 * a1[j] + cz)
                s1.append(1.0 / (t + 1.0))

            out = b3
            for i in range(NUM_MIX):
                acc = b2[i]
                for j in range(NUM_MIX):
                    acc = acc + row(_W2_OFF + i * NUM_MIX + j) * s1[j]
                t = jnp.exp(acc)
                out = out + w3[i] * (1.0 / (t + 1.0))

            out_v[pl.ds(ob + o, LANES)] = jnp.where(is_std, jnp.exp(out), out)

        return carry

    lax.fori_loop(0, 2, run_net, 0)

    pltpu.sync_copy(out_v.at[pl.ds(0, CHUNK)], mean_hbm.at[pl.ds(base, CHUNK)])
    pltpu.sync_copy(out_v.at[pl.ds(CHUNK, CHUNK)],
                    std_hbm.at[pl.ds(base, CHUNK)])


def _scratch_types():
    return [
        pltpu.VMEM((CHUNK,), jnp.float32),
        pltpu.VMEM((CHUNK,), jnp.int32),
        pltpu.VMEM((2 * _NET_F,), jnp.float32),
        pltpu.VMEM((2 * _ROWS_F,), jnp.float32),
        pltpu.VMEM((2 * CHUNK,), jnp.float32),
        pltpu.SemaphoreType.DMA,
    ]


@functools.cache
def _sc_call():
    return functools.partial(
        pl.kernel,
        out_type=(
            jax.ShapeDtypeStruct((N,), jnp.float32),
            jax.ShapeDtypeStruct((N,), jnp.float32),
        ),
        mesh=plsc.VectorSubcoreMesh(
            core_axis_name="c", subcore_axis_name="s",
            num_cores=NC, num_subcores=NS,
        ),
        scratch_types=_scratch_types(),
        compiler_params=pltpu.CompilerParams(
            needs_layout_passes=False,
            disable_bounds_checks=True,
            disable_semaphore_checks=True,
        ),
    )(_sc_body)


def _pack_net(W1, b1, W2, b2, W3, b3):
    return jnp.concatenate([
        (-2.0 * (W1[:, 1:].T + b1[None, :])).reshape(-1),
        -2.0 * W1[:, 0],
        (-4.0 * W2).reshape(-1),
        -2.0 * b2 + 2.0 * W2.sum(axis=1),
        2.0 * W3[0],
        b3 - W3[0].sum(),
    ])


def kernel(obs, k, z, mW1, mb1, mW2, mb2, mW3, mb3,
           sW1, sb1, sW2, sb2, sW3, sb3):
    del k
    cw = jnp.concatenate([
        _pack_net(mW1, mb1, mW2, mb2, mW3, mb3),
        _pack_net(sW1, sb1, sW2, sb2, sW3, sb3),
    ])
    mean, std = _sc_call()(obs, z.astype(jnp.int32), cw)
    return mean, std
